# Initial kernel scaffold; baseline (speedup 1.0000x reference)
import jax
import jax.numpy as jnp
from jax.experimental import pallas as pl


def kernel(x, expert_sel, keys_w, values_w):
    raise NotImplementedError("write your pallas kernel here")


# speedup vs baseline: 1.0007x; 1.0007x over previous
import functools

import jax
import jax.numpy as jnp
from jax.experimental import pallas as pl
from jax.experimental.pallas import tpu as pltpu

DMODEL = 1024
N_EXPERTS = 8
EXPERT_SIZE = 512
N_HEADS = 2
T = 2048


def _moe_dense_kernel(x_ref, sel_ref, keys_ref, values_ref, out_ref, w_ref):
    e = pl.program_id(0)

    @pl.when(e == 0)
    def _router():
        x = x_ref[...]
        logits = jax.lax.dot_general(
            x, sel_ref[...],
            (((1,), (1,)), ((), ())),
            preferred_element_type=jnp.float32,
        )
        idx = jax.lax.broadcasted_iota(jnp.int32, logits.shape, 1)
        m1 = jnp.max(logits, axis=1, keepdims=True)
        i1 = jnp.min(jnp.where(logits == m1, idx, N_EXPERTS), axis=1, keepdims=True)
        oh1 = idx == i1
        rest = jnp.where(oh1, -jnp.inf, logits)
        m2 = jnp.max(rest, axis=1, keepdims=True)
        i2 = jnp.min(jnp.where(rest == m2, idx, N_EXPERTS), axis=1, keepdims=True)
        oh2 = idx == i2
        w_ref[...] = jax.nn.sigmoid(logits) * (oh1 | oh2).astype(jnp.float32)

    x = x_ref[...]
    scores = jax.lax.dot_general(
        x, keys_ref[0],
        (((1,), (0,)), ((), ())),
        preferred_element_type=jnp.float32,
    )
    w_all = w_ref[...]
    lane = jax.lax.broadcasted_iota(jnp.int32, w_all.shape, 1)
    w_col = jnp.sum(jnp.where(lane == e, w_all, 0.0), axis=1, keepdims=True)
    h = jnp.maximum(scores, 0.0) * w_col
    contrib = jax.lax.dot_general(
        h, values_ref[0],
        (((1,), (0,)), ((), ())),
        preferred_element_type=jnp.float32,
    )

    @pl.when(e == 0)
    def _init():
        out_ref[...] = contrib

    @pl.when(e != 0)
    def _acc():
        out_ref[...] = out_ref[...] + contrib


@jax.jit
def kernel(x, expert_sel, keys_w, values_w):
    return pl.pallas_call(
        _moe_dense_kernel,
        grid=(N_EXPERTS,),
        in_specs=[
            pl.BlockSpec((T, DMODEL), lambda e: (0, 0)),
            pl.BlockSpec((N_EXPERTS, DMODEL), lambda e: (0, 0)),
            pl.BlockSpec((1, DMODEL, EXPERT_SIZE), lambda e: (e, 0, 0)),
            pl.BlockSpec((1, EXPERT_SIZE, DMODEL), lambda e: (e, 0, 0)),
        ],
        out_specs=pl.BlockSpec((T, DMODEL), lambda e: (0, 0)),
        out_shape=jax.ShapeDtypeStruct((T, DMODEL), jnp.float32),
        scratch_shapes=[pltpu.VMEM((T, N_EXPERTS), jnp.float32)],
    )(x, expert_sel, keys_w, values_w)
